# Initial kernel scaffold; baseline (speedup 1.0000x reference)
import functools

import jax
import jax.numpy as jnp
from jax import lax
from jax.experimental import pallas as pl
from jax.experimental.pallas import tpu as pltpu
from jax.experimental.pallas import tpu_sc as plsc

_B, _T, _D = 4096, 200, 64
_N = _B * _T
_NW = 32
_NPW = _N // _NW
_CH = 128
_NCH = _NPW // _CH

_mesh = plsc.VectorSubcoreMesh(core_axis_name="c", subcore_axis_name="s")


@functools.partial(
    pl.kernel,
    out_type=jax.ShapeDtypeStruct((_N, _D), jnp.float32),
    mesh=_mesh,
    scratch_types=[
        pltpu.VMEM((_NCH, _CH), jnp.int32),
        pltpu.VMEM((_CH, _D), jnp.float32),
        pltpu.SemaphoreType.DMA,
    ],
)
def _emb_gather(x_hbm, tab_hbm, out_hbm, idx_v, rows_v, gsem):
    wid = lax.axis_index("s") * 2 + lax.axis_index("c")
    base = wid * _NPW
    pltpu.sync_copy(x_hbm.at[pl.ds(wid * _NCH, _NCH), :], idx_v)

    def body(c, carry):
        pltpu.async_copy(tab_hbm.at[idx_v.at[c]], rows_v, gsem).wait()
        pltpu.sync_copy(rows_v, out_hbm.at[pl.ds(base + c * _CH, _CH), :])
        return carry

    lax.fori_loop(0, _NCH, body, 0)


def kernel(x, table):
    xf = x.reshape(_N // _CH, _CH).astype(jnp.int32)
    out = _emb_gather(xf, table)
    return out.reshape(_B, _T, _D)


# speedup vs baseline: 4.1375x; 4.1375x over previous
import functools

import jax
import jax.numpy as jnp
from jax import lax
from jax.experimental import pallas as pl
from jax.experimental.pallas import tpu as pltpu
from jax.experimental.pallas import tpu_sc as plsc

_B, _T, _D = 4096, 200, 64
_N = _B * _T
_NW = 32
_NPW = _N // _NW
_CH = 128
_NCH = _NPW // _CH

_mesh = plsc.VectorSubcoreMesh(core_axis_name="c", subcore_axis_name="s")


@functools.partial(
    pl.kernel,
    out_type=jax.ShapeDtypeStruct((_N, _D), jnp.float32),
    mesh=_mesh,
    scratch_types=[
        pltpu.VMEM((_NCH, _CH), jnp.int32),
        pltpu.VMEM((_CH, _D), jnp.float32),
        pltpu.SemaphoreType.DMA,
    ],
    compiler_params=pltpu.CompilerParams(use_tc_tiling_on_sc=False),
)
def _emb_gather(x_hbm, tab_hbm, out_hbm, idx_v, rows_v, gsem):
    wid = lax.axis_index("s") * 2 + lax.axis_index("c")
    base = wid * _NPW
    pltpu.sync_copy(x_hbm.at[pl.ds(wid * _NCH, _NCH), :], idx_v)

    def body(c, carry):
        pltpu.async_copy(tab_hbm.at[idx_v.at[c]], rows_v, gsem).wait()
        pltpu.sync_copy(rows_v, out_hbm.at[pl.ds(base + c * _CH, _CH), :])
        return carry

    lax.fori_loop(0, _NCH, body, 0)


def kernel(x, table):
    xf = x.reshape(_N // _CH, _CH).astype(jnp.int32)
    out = _emb_gather(xf, table)
    return out.reshape(_B, _T, _D)


# speedup vs baseline: 4.9070x; 1.1860x over previous
import functools

import jax
import jax.numpy as jnp
from jax import lax
from jax.experimental import pallas as pl
from jax.experimental.pallas import tpu as pltpu
from jax.experimental.pallas import tpu_sc as plsc

_B, _T, _D = 4096, 200, 64
_N = _B * _T
_NW = 32
_NPW = _N // _NW
_CH = 128
_NCH = _NPW // _CH
_K = 4
_NSUP = _NCH // _K

_mesh = plsc.VectorSubcoreMesh(core_axis_name="c", subcore_axis_name="s")


@functools.partial(
    pl.kernel,
    out_type=jax.ShapeDtypeStruct((_N, _D), jnp.float32),
    mesh=_mesh,
    scratch_types=[
        pltpu.VMEM((_NCH, _CH), jnp.int32),
        pltpu.VMEM((2 * _K, _CH, _D), jnp.float32),
        pltpu.SemaphoreType.DMA,
        pltpu.SemaphoreType.DMA,
    ],
    compiler_params=pltpu.CompilerParams(use_tc_tiling_on_sc=False),
)
def _emb_gather(x_hbm, tab_hbm, out_hbm, idx_v, rows_v, gsem, ssem):
    wid = lax.axis_index("s") * 2 + lax.axis_index("c")
    base = wid * _NPW
    pltpu.sync_copy(x_hbm.at[pl.ds(wid * _NCH, _NCH), :], idx_v)

    def fire_gather(c, buf):
        pltpu.async_copy(tab_hbm.at[idx_v.at[c]], rows_v.at[buf], gsem)

    def fire_scatter(c, buf):
        pltpu.async_copy(
            rows_v.at[buf], out_hbm.at[pl.ds(base + c * _CH, _CH), :], ssem)

    def drain_gathers(k):
        for _ in range(k):
            pltpu.make_async_copy(
                tab_hbm.at[idx_v.at[0]], rows_v.at[0], gsem).wait()

    def drain_scatters(k):
        for _ in range(k):
            pltpu.make_async_copy(
                rows_v.at[0], out_hbm.at[pl.ds(base, _CH), :], ssem).wait()

    for b in range(_K):
        fire_gather(b, b)

    def sbody(t, carry):
        s0 = 2 * t
        drain_gathers(_K)

        @pl.when(t > 0)
        def _():
            drain_scatters(_K)

        for b in range(_K):
            fire_gather((s0 + 1) * _K + b, _K + b)
        for b in range(_K):
            fire_scatter(s0 * _K + b, b)

        drain_gathers(_K)
        drain_scatters(_K)

        @pl.when(s0 + 2 < _NSUP)
        def _():
            for b in range(_K):
                fire_gather((s0 + 2) * _K + b, b)

        for b in range(_K):
            fire_scatter((s0 + 1) * _K + b, _K + b)
        return carry

    lax.fori_loop(0, _NSUP // 2, sbody, 0)
    drain_scatters(_K)


def kernel(x, table):
    xf = x.reshape(_N // _CH, _CH).astype(jnp.int32)
    out = _emb_gather(xf, table)
    return out.reshape(_B, _T, _D)


# speedup vs baseline: 5.5696x; 1.1350x over previous
import functools

import jax
import jax.numpy as jnp
from jax import lax
from jax.experimental import pallas as pl
from jax.experimental.pallas import tpu as pltpu
from jax.experimental.pallas import tpu_sc as plsc

_B, _T, _D = 4096, 200, 64
_N = _B * _T
_NW = 32
_NPW = _N // _NW
_CH = 128
_NCH = _NPW // _CH
_K = 4
_NSUP = _NCH // _K

_mesh = plsc.VectorSubcoreMesh(core_axis_name="c", subcore_axis_name="s")


@functools.partial(
    pl.kernel,
    out_type=jax.ShapeDtypeStruct((_N, _D), jnp.float32),
    mesh=_mesh,
    scratch_types=[
        pltpu.VMEM((_NCH, _CH), jnp.int32),
        pltpu.VMEM((2 * _K, _CH, _D), jnp.float32),
        pltpu.VMEM_SHARED((8192, _D), jnp.float32),
        pltpu.SemaphoreType.DMA,
        pltpu.SemaphoreType.DMA,
    ],
    compiler_params=pltpu.CompilerParams(use_tc_tiling_on_sc=False),
)
def _emb_gather(x_hbm, tab_hbm, out_hbm, idx_v, rows_v, tab_sh, gsem, ssem):
    sid = lax.axis_index("s")
    wid = sid * 2 + lax.axis_index("c")
    base = wid * _NPW
    pltpu.sync_copy(x_hbm.at[pl.ds(wid * _NCH, _NCH), :], idx_v)
    _ROWS_PER_SUB = 8192 // 16
    pltpu.sync_copy(tab_hbm.at[pl.ds(sid * _ROWS_PER_SUB, _ROWS_PER_SUB), :],
                    tab_sh.at[pl.ds(sid * _ROWS_PER_SUB, _ROWS_PER_SUB), :])
    plsc.subcore_barrier()

    def fire_gather(c, buf):
        pltpu.async_copy(tab_sh.at[idx_v.at[c]], rows_v.at[buf], gsem)

    def fire_scatter(c, buf):
        pltpu.async_copy(
            rows_v.at[buf], out_hbm.at[pl.ds(base + c * _CH, _CH), :], ssem)

    def drain_gathers(k):
        for _ in range(k):
            pltpu.make_async_copy(
                tab_sh.at[idx_v.at[0]], rows_v.at[0], gsem).wait()

    def drain_scatters(k):
        for _ in range(k):
            pltpu.make_async_copy(
                rows_v.at[0], out_hbm.at[pl.ds(base, _CH), :], ssem).wait()

    for b in range(_K):
        fire_gather(b, b)

    def sbody(t, carry):
        s0 = 2 * t
        drain_gathers(_K)

        @pl.when(t > 0)
        def _():
            drain_scatters(_K)

        for b in range(_K):
            fire_gather((s0 + 1) * _K + b, _K + b)
        for b in range(_K):
            fire_scatter(s0 * _K + b, b)

        drain_gathers(_K)
        drain_scatters(_K)

        @pl.when(s0 + 2 < _NSUP)
        def _():
            for b in range(_K):
                fire_gather((s0 + 2) * _K + b, b)

        for b in range(_K):
            fire_scatter((s0 + 1) * _K + b, _K + b)
        return carry

    lax.fori_loop(0, _NSUP // 2, sbody, 0)
    drain_scatters(_K)


def kernel(x, table):
    xf = x.reshape(_N // _CH, _CH).astype(jnp.int32)
    out = _emb_gather(xf, table)
    return out.reshape(_B, _T, _D)


# speedup vs baseline: 5.5978x; 1.0051x over previous
import functools

import jax
import jax.numpy as jnp
from jax import lax
from jax.experimental import pallas as pl
from jax.experimental.pallas import tpu as pltpu
from jax.experimental.pallas import tpu_sc as plsc

_B, _T, _D = 4096, 200, 64
_N = _B * _T
_NW = 32
_BPW = _B // _NW
_NPW = _N // _NW
_G = 40
_NG = _T // _G
_K = 2
_NSUP = _BPW // _K

_mesh = plsc.VectorSubcoreMesh(core_axis_name="c", subcore_axis_name="s")


@functools.partial(
    pl.kernel,
    out_type=jax.ShapeDtypeStruct((_B, _T, _D), jnp.float32),
    mesh=_mesh,
    scratch_types=[
        pltpu.VMEM((_NPW,), jnp.int32),
        pltpu.VMEM((2 * _K, _T, _D), jnp.float32),
        pltpu.VMEM_SHARED((8192, _D), jnp.float32),
        pltpu.SemaphoreType.DMA,
        pltpu.SemaphoreType.DMA,
    ],
    compiler_params=pltpu.CompilerParams(use_tc_tiling_on_sc=False),
)
def _emb_gather(x_hbm, tab_hbm, out_hbm, idx_v, rows_v, tab_sh, gsem, ssem):
    sid = lax.axis_index("s")
    wid = sid * 2 + lax.axis_index("c")
    pltpu.sync_copy(x_hbm.at[pl.ds(wid * _NPW, _NPW)], idx_v)
    _RPS = 8192 // 16
    pltpu.sync_copy(tab_hbm.at[pl.ds(sid * _RPS, _RPS), :],
                    tab_sh.at[pl.ds(sid * _RPS, _RPS), :])
    plsc.subcore_barrier()

    def fire_gather(c, buf):
        for k in range(_NG):
            pltpu.async_copy(
                tab_sh.at[idx_v.at[pl.ds(c * _T + k * _G, _G)]],
                rows_v.at[buf, pl.ds(k * _G, _G), :], gsem)

    def fire_scatter(c, buf):
        pltpu.async_copy(rows_v.at[buf], out_hbm.at[wid * _BPW + c], ssem)

    def drain_gathers(k):
        for _ in range(_NG * k):
            pltpu.make_async_copy(
                tab_sh.at[idx_v.at[pl.ds(0, _G)]],
                rows_v.at[0, pl.ds(0, _G), :], gsem).wait()

    def drain_scatters(k):
        for _ in range(k):
            pltpu.make_async_copy(
                rows_v.at[0], out_hbm.at[0], ssem).wait()

    for b in range(_K):
        fire_gather(b, b)

    def sbody(t, carry):
        s0 = 2 * t
        drain_gathers(_K)

        @pl.when(t > 0)
        def _():
            drain_scatters(_K)

        for b in range(_K):
            fire_gather((s0 + 1) * _K + b, _K + b)
        for b in range(_K):
            fire_scatter(s0 * _K + b, b)

        drain_gathers(_K)
        drain_scatters(_K)

        @pl.when(s0 + 2 < _NSUP)
        def _():
            for b in range(_K):
                fire_gather((s0 + 2) * _K + b, b)

        for b in range(_K):
            fire_scatter((s0 + 1) * _K + b, _K + b)
        return carry

    lax.fori_loop(0, _NSUP // 2, sbody, 0)
    drain_scatters(_K)


def kernel(x, table):
    return _emb_gather(x.reshape(_N).astype(jnp.int32), table)
